# baseline (speedup 1.0000x reference)
import functools

import jax
import jax.numpy as jnp
from jax import lax
from jax.experimental import pallas as pl
from jax.experimental.pallas import tpu as pltpu
from jax.experimental.pallas import tpu_sc as plsc

SMOOTHING_VALUE = 0.1
CONFIDENCE_VALUE = 1.0 - SMOOTHING_VALUE
PAD_TOKEN_ID = 0
TRG_VOCAB_SIZE = 32000

BATCH = 4096
BR = 256
BC = 6400

NUM_CORES = 2
NUM_SUBCORES = 16
LANES = 16
NUM_WORKERS = NUM_CORES * NUM_SUBCORES
ROWS_PER_WORKER = BATCH // NUM_WORKERS


def _base_fill_kernel(tok_ref, out_ref):
    fill = SMOOTHING_VALUE / (TRG_VOCAB_SIZE - 2)
    col = jax.lax.broadcasted_iota(jnp.int32, (BR, BC), 1) + pl.program_id(1) * BC
    t = tok_ref[:, 0][:, None]
    val = jnp.where((col == PAD_TOKEN_ID) | (t == PAD_TOKEN_ID), 0.0, fill)
    out_ref[...] = val


def _sc_scatter_body(tok_hbm, out_flat_ref, tok_v, idx_v, val_v, sem):
    wid = lax.axis_index("s") * NUM_CORES + lax.axis_index("c")
    row0 = wid * ROWS_PER_WORKER
    pltpu.sync_copy(tok_hbm.at[pl.ds(row0, ROWS_PER_WORKER)], tok_v)
    lane = lax.iota(jnp.int32, LANES)
    for i in range(ROWS_PER_WORKER // LANES):
        t = tok_v[pl.ds(i * LANES, LANES)]
        row = row0 + i * LANES + lane
        idx_v[pl.ds(i * LANES, LANES)] = row * TRG_VOCAB_SIZE + t
        val_v[pl.ds(i * LANES, LANES)] = jnp.where(
            t == PAD_TOKEN_ID, 0.0, CONFIDENCE_VALUE)
    pltpu.async_copy(val_v, out_flat_ref.at[idx_v], sem).wait()


_sc_scatter = pl.kernel(
    _sc_scatter_body,
    mesh=plsc.VectorSubcoreMesh(
        core_axis_name="c", subcore_axis_name="s",
        num_cores=NUM_CORES, num_subcores=NUM_SUBCORES),
    scratch_types=[
        pltpu.VMEM((ROWS_PER_WORKER,), jnp.int32),
        pltpu.VMEM((ROWS_PER_WORKER,), jnp.int32),
        pltpu.VMEM((ROWS_PER_WORKER,), jnp.float32),
        pltpu.SemaphoreType.DMA,
    ],
)


def kernel(trg_token_ids_batch):
    b = trg_token_ids_batch.shape[0]
    tok = trg_token_ids_batch.astype(jnp.int32)
    base = pl.pallas_call(
        _base_fill_kernel,
        grid=(b // BR, TRG_VOCAB_SIZE // BC),
        in_specs=[pl.BlockSpec((BR, 1), lambda i, j: (i, 0))],
        out_specs=pl.BlockSpec((BR, BC), lambda i, j: (i, j)),
        out_shape=jax.ShapeDtypeStruct((b, TRG_VOCAB_SIZE), jnp.float32),
    )(tok)
    out_ref = jax.new_ref(base.reshape(b * TRG_VOCAB_SIZE))
    _sc_scatter(tok[:, 0], out_ref)
    return jax.freeze(out_ref).reshape(b, TRG_VOCAB_SIZE)
